# baseline (speedup 1.0000x reference)
import jax
import jax.numpy as jnp
from jax import lax
from jax.experimental import pallas as pl
from jax.experimental.pallas import tpu as pltpu
from jax.experimental.pallas import tpu_sc as plsc

B, T, N, C = 16, 12, 2048, 4
H = 80
TODC = 24
DHC = 8
ADPC = 40
OUTC = 152
STEPS = 288
CH = 128
NCH = N // CH
NW = 32
COMBOS = T * NCH
PER_W = COMBOS // NW
NK = PER_W * B


def _body(x3, w, b, tod2d, dh2d, adp, out, w_v, b_v,
          x_v2, h_o2, tod_o2, dh_o2, adp_v2, tidx2, didx2,
          sem_in, sem_out, sem_ain, sem_aout, sem_g):
    cc = lax.axis_index("c")
    ss = lax.axis_index("s")
    wid = ss * 2 + cc

    pltpu.sync_copy(w, w_v)
    pltpu.sync_copy(b, b_v)
    ws = [[w_v[d, pl.ds(j * 16, 16)] for j in range(5)] for d in range(4)]
    bs = [b_v[pl.ds(j * 16, 16)] for j in range(5)]

    lane = lax.iota(jnp.int32, 16)
    cidx = [jnp.full((16,), i, jnp.int32) for i in range(8)]

    def chunk_coords(k):
        ci = k // B
        bb = k % B
        cid = wid * PER_W + ci
        t = cid // NCH
        n0 = (cid % NCH) * CH
        return ci, bb, t, n0

    def issue_adp(ci):
        cid = wid * PER_W + ci
        t = cid // NCH
        n0 = (cid % NCH) * CH
        pltpu.async_copy(adp.at[t, pl.ds(n0, CH), :],
                         adp_v2.at[ci % 2], sem_ain.at[ci % 2])

    def out_descs(slot, t, bb, n0):
        ns = pl.ds(n0, CH)
        return [
            pltpu.make_async_copy(h_o2.at[slot],
                                  out.at[bb, t, ns, pl.ds(0, H)],
                                  sem_out.at[slot]),
            pltpu.make_async_copy(tod_o2.at[slot],
                                  out.at[bb, t, ns, pl.ds(H, TODC)],
                                  sem_out.at[slot]),
            pltpu.make_async_copy(dh_o2.at[slot],
                                  out.at[bb, t, ns, pl.ds(H + TODC, DHC)],
                                  sem_out.at[slot]),
        ]

    def adp_desc(aslot, t, bb, n0):
        return pltpu.make_async_copy(
            adp_v2.at[aslot],
            out.at[bb, t, pl.ds(n0, CH), pl.ds(H + TODC + DHC, ADPC)],
            sem_aout.at[aslot])

    issue_adp(0)
    pltpu.async_copy(x3.at[0, wid * PER_W // NCH,
                           pl.ds((wid * PER_W % NCH) * CH * C, CH * C)],
                     x_v2.at[0], sem_in.at[0])

    def chunk_body(k, carry):
        ci, bb, t, n0 = chunk_coords(k)
        slot = k % 2
        aslot = ci % 2

        @pl.when(bb == 0)
        def _():
            pltpu.make_async_copy(adp.at[t, pl.ds(n0, CH), :],
                                  adp_v2.at[aslot],
                                  sem_ain.at[aslot]).wait()

            @pl.when(ci >= 1)
            def _():
                d = adp_desc(1 - aslot, t, bb, n0)
                for _i in range(B):
                    d.wait()

            @pl.when(ci + 1 < PER_W)
            def _():
                issue_adp(ci + 1)

        ci2, bb2, t2, n02 = chunk_coords(k + 1)
        pltpu.make_async_copy(x3.at[bb, t, pl.ds(n0 * C, CH * C)],
                              x_v2.at[slot], sem_in.at[slot]).wait()

        @pl.when(k + 1 < NK)
        def _():
            pltpu.async_copy(x3.at[bb2, t2, pl.ds(n02 * C, CH * C)],
                             x_v2.at[1 - slot], sem_in.at[1 - slot])

        @pl.when(k >= 2)
        def _():
            for d in out_descs(slot, t, bb, n0):
                d.wait()

        x_v = x_v2.at[slot]

        def g_body(g, c3):
            rows = g * 16 + lane
            xoff = rows * C
            tod_f = plsc.load_gather(x_v, [xoff + 1])
            dow_f = plsc.load_gather(x_v, [xoff + 2])
            hol_f = plsc.load_gather(x_v, [xoff + 3])
            tod_i = jnp.clip((tod_f * float(STEPS)).astype(jnp.int32),
                             0, STEPS - 1)
            dh_i = (jnp.clip(dow_f.astype(jnp.int32), 0, 1) * 2
                    + jnp.clip(hol_f.astype(jnp.int32), 0, 1))
            tidx2[slot, pl.ds(g * 16, 16)] = tod_i
            didx2[slot, pl.ds(g * 16, 16)] = dh_i
            return c3
        lax.fori_loop(0, CH // 16, g_body, 0)

        gt = pltpu.async_copy(tod2d.at[tidx2.at[slot]], tod_o2.at[slot],
                              sem_g.at[slot])
        gd = pltpu.async_copy(dh2d.at[didx2.at[slot]], dh_o2.at[slot],
                              sem_g.at[slot])

        def h_body(p2):
            xx = x_v[pl.ds(p2 * 8, 16)]
            for pt in range(2):
                p = p2 * 2 + pt
                bc = [jnp.take_along_axis(xx, cidx[pt * 4 + d], axis=0)
                      for d in range(4)]
                for j in range(5):
                    acc = bs[j] + bc[0] * ws[0][j]
                    acc = acc + bc[1] * ws[1][j]
                    acc = acc + bc[2] * ws[2][j]
                    acc = acc + bc[3] * ws[3][j]
                    h_o2[slot, p, pl.ds(j * 16, 16)] = acc
        plsc.parallel_loop(0, CH // 2, 1, unroll=2)(h_body)

        gt.wait()
        gd.wait()

        ns = pl.ds(n0, CH)
        pltpu.async_copy(h_o2.at[slot], out.at[bb, t, ns, pl.ds(0, H)],
                         sem_out.at[slot])
        pltpu.async_copy(tod_o2.at[slot], out.at[bb, t, ns, pl.ds(H, TODC)],
                         sem_out.at[slot])
        pltpu.async_copy(dh_o2.at[slot],
                         out.at[bb, t, ns, pl.ds(H + TODC, DHC)],
                         sem_out.at[slot])
        pltpu.async_copy(adp_v2.at[aslot],
                         out.at[bb, t, ns, pl.ds(H + TODC + DHC, ADPC)],
                         sem_aout.at[aslot])
        return carry

    lax.fori_loop(0, NK, chunk_body, 0)

    for k in (NK - 2, NK - 1):
        ci, bb, t, n0 = chunk_coords(k)
        for d in out_descs(k % 2, t, bb, n0):
            d.wait()
    ci, bb, t, n0 = chunk_coords(NK - 1)
    d = adp_desc((PER_W - 1) % 2, t, bb, n0)
    for _i in range(B):
        d.wait()


def kernel(x, W_in, b_in, tod_table, dow_table, hol_table, adaptive_emb):
    x3 = x.reshape(B, T, N * C)
    dh2d = jnp.concatenate(
        [jnp.repeat(dow_table, 2, axis=0), jnp.tile(hol_table, (2, 1))],
        axis=1)

    mesh = plsc.VectorSubcoreMesh(core_axis_name="c", subcore_axis_name="s")
    run = pl.kernel(
        _body,
        out_type=jax.ShapeDtypeStruct((B, T, N, OUTC), jnp.float32),
        mesh=mesh,
        compiler_params=pltpu.CompilerParams(use_tc_tiling_on_sc=False,
                                             needs_layout_passes=False),
        scratch_types=[
            pltpu.VMEM((C, H), jnp.float32),
            pltpu.VMEM((H,), jnp.float32),
            pltpu.VMEM((2, CH * C), jnp.float32),
            pltpu.VMEM((2, CH, H), jnp.float32),
            pltpu.VMEM((2, CH, TODC), jnp.float32),
            pltpu.VMEM((2, CH, DHC), jnp.float32),
            pltpu.VMEM((2, CH, ADPC), jnp.float32),
            pltpu.VMEM((2, CH), jnp.int32),
            pltpu.VMEM((2, CH), jnp.int32),
            pltpu.SemaphoreType.DMA((2,)),
            pltpu.SemaphoreType.DMA((2,)),
            pltpu.SemaphoreType.DMA((2,)),
            pltpu.SemaphoreType.DMA((2,)),
            pltpu.SemaphoreType.DMA((2,)),
        ],
    )
    return run(x3, W_in, b_in, tod_table, dh2d, adaptive_emb)


# speedup vs baseline: 2.7246x; 2.7246x over previous
import jax
import jax.numpy as jnp
from jax import lax
from jax.experimental import pallas as pl
from jax.experimental.pallas import tpu as pltpu
from jax.experimental.pallas import tpu_sc as plsc

B, T, N, C = 16, 12, 2048, 4
H = 80
TODC = 24
DHC = 8
ADPC = 40
OUTC = 152
STEPS = 288
CH = 128
NCH = N // CH
NW = 32
COMBOS = T * NCH
PER_W = COMBOS // NW
NK = PER_W * B


def _body(x3, w, b, todf, dhf, adp, out, w_v, b_v, tod_v, dh_v,
          x_v2, h_o2, tod_o2, dh_o2, adp_v2,
          sem_in, sem_out, sem_ain, sem_aout):
    cc = lax.axis_index("c")
    ss = lax.axis_index("s")
    wid = ss * 2 + cc

    pltpu.sync_copy(w, w_v)
    pltpu.sync_copy(b, b_v)
    pltpu.sync_copy(todf, tod_v)
    pltpu.sync_copy(dhf, dh_v)

    lane = lax.iota(jnp.int32, 16)
    cidx = [jnp.full((16,), i, jnp.int32) for i in range(8)]
    ws = [[w_v[d, pl.ds(j * 16, 16)] for j in range(5)] for d in range(4)]
    bs = [b_v[pl.ds(j * 16, 16)] for j in range(5)]

    def chunk_coords(k):
        ci = k // B
        bb = k % B
        cid = wid * PER_W + ci
        t = cid // NCH
        n0 = (cid % NCH) * CH
        return ci, bb, t, n0

    def issue_x(k):
        ci, bb, t, n0 = chunk_coords(k)
        slot = k % 2
        pltpu.async_copy(x3.at[bb, t, pl.ds(n0 * C, CH * C)],
                         x_v2.at[slot], sem_in.at[slot])

    def issue_adp(ci):
        cid = wid * PER_W + ci
        t = cid // NCH
        n0 = (cid % NCH) * CH
        pltpu.async_copy(adp.at[t, pl.ds(n0, CH), :],
                         adp_v2.at[ci % 2], sem_ain.at[ci % 2])

    def out_descs(slot, t, bb, n0):
        ns = pl.ds(n0, CH)
        return [
            pltpu.make_async_copy(h_o2.at[slot],
                                  out.at[bb, t, ns, pl.ds(0, H)],
                                  sem_out.at[slot]),
            pltpu.make_async_copy(tod_o2.at[slot],
                                  out.at[bb, t, ns, pl.ds(H, TODC)],
                                  sem_out.at[slot]),
            pltpu.make_async_copy(dh_o2.at[slot],
                                  out.at[bb, t, ns, pl.ds(H + TODC, DHC)],
                                  sem_out.at[slot]),
        ]

    def adp_desc(aslot, t, bb, n0):
        return pltpu.make_async_copy(
            adp_v2.at[aslot],
            out.at[bb, t, pl.ds(n0, CH), pl.ds(H + TODC + DHC, ADPC)],
            sem_aout.at[aslot])

    issue_adp(0)
    issue_x(0)

    def chunk_body(k, carry):
        ci, bb, t, n0 = chunk_coords(k)
        slot = k % 2
        aslot = ci % 2

        @pl.when(bb == 0)
        def _():
            pltpu.make_async_copy(adp.at[t, pl.ds(n0, CH), :],
                                  adp_v2.at[aslot],
                                  sem_ain.at[aslot]).wait()

            @pl.when(ci >= 1)
            def _():
                d = adp_desc(1 - aslot, t, bb, n0)
                for _i in range(B):
                    d.wait()

            @pl.when(ci + 1 < PER_W)
            def _():
                issue_adp(ci + 1)

        ci2, bb2, t2, n02 = chunk_coords(k + 1)
        pltpu.make_async_copy(x3.at[bb, t, pl.ds(n0 * C, CH * C)],
                              x_v2.at[slot], sem_in.at[slot]).wait()

        @pl.when(k + 1 < NK)
        def _():
            pltpu.async_copy(x3.at[bb2, t2, pl.ds(n02 * C, CH * C)],
                             x_v2.at[1 - slot], sem_in.at[1 - slot])

        @pl.when(k >= 2)
        def _():
            for d in out_descs(slot, t, bb, n0):
                d.wait()

        x_v = x_v2.at[slot]

        def h_body(p2):
            xx = x_v[pl.ds(p2 * 8, 16)]
            for pt in range(2):
                p = p2 * 2 + pt
                bc = [jnp.take_along_axis(xx, cidx[pt * 4 + d], axis=0)
                      for d in range(4)]
                for j in range(5):
                    acc = bs[j] + bc[0] * ws[0][j]
                    acc = acc + bc[1] * ws[1][j]
                    acc = acc + bc[2] * ws[2][j]
                    acc = acc + bc[3] * ws[3][j]
                    h_o2[slot, p, pl.ds(j * 16, 16)] = acc
        plsc.parallel_loop(0, CH // 2, 1, unroll=2)(h_body)

        def g_body(g):
            rows = g * 16 + lane
            xoff = rows * C
            tod_f = plsc.load_gather(x_v, [xoff + 1])
            dow_f = plsc.load_gather(x_v, [xoff + 2])
            hol_f = plsc.load_gather(x_v, [xoff + 3])
            tod_i = jnp.clip((tod_f * float(STEPS)).astype(jnp.int32),
                             0, STEPS - 1)
            toff = tod_i * TODC
            dh_i = (jnp.clip(dow_f.astype(jnp.int32), 0, 1) * 2
                    + jnp.clip(hol_f.astype(jnp.int32), 0, 1))
            dhoff = dh_i * DHC
            for ch in range(TODC):
                v = plsc.load_gather(tod_v, [toff + ch])
                plsc.store_scatter(
                    tod_o2.at[slot],
                    [rows, jnp.full((16,), ch, jnp.int32)], v)
            for ch in range(DHC):
                v = plsc.load_gather(dh_v, [dhoff + ch])
                plsc.store_scatter(
                    dh_o2.at[slot],
                    [rows, jnp.full((16,), ch, jnp.int32)], v)
        plsc.parallel_loop(0, CH // 16, 1, unroll=1)(g_body)

        ns = pl.ds(n0, CH)
        pltpu.async_copy(h_o2.at[slot], out.at[bb, t, ns, pl.ds(0, H)],
                         sem_out.at[slot])
        pltpu.async_copy(tod_o2.at[slot], out.at[bb, t, ns, pl.ds(H, TODC)],
                         sem_out.at[slot])
        pltpu.async_copy(dh_o2.at[slot],
                         out.at[bb, t, ns, pl.ds(H + TODC, DHC)],
                         sem_out.at[slot])
        pltpu.async_copy(adp_v2.at[aslot],
                         out.at[bb, t, ns, pl.ds(H + TODC + DHC, ADPC)],
                         sem_aout.at[aslot])
        return carry

    lax.fori_loop(0, NK, chunk_body, 0)

    for k in (NK - 2, NK - 1):
        ci, bb, t, n0 = chunk_coords(k)
        for d in out_descs(k % 2, t, bb, n0):
            d.wait()
    ci, bb, t, n0 = chunk_coords(NK - 1)
    d = adp_desc((PER_W - 1) % 2, t, bb, n0)
    for _i in range(B):
        d.wait()


def kernel(x, W_in, b_in, tod_table, dow_table, hol_table, adaptive_emb):
    x3 = x.reshape(B, T, N * C)
    todf = tod_table.reshape(-1)
    dhf = jnp.concatenate(
        [jnp.repeat(dow_table, 2, axis=0), jnp.tile(hol_table, (2, 1))],
        axis=1).reshape(-1)

    mesh = plsc.VectorSubcoreMesh(core_axis_name="c", subcore_axis_name="s")
    run = pl.kernel(
        _body,
        out_type=jax.ShapeDtypeStruct((B, T, N, OUTC), jnp.float32),
        mesh=mesh,
        compiler_params=pltpu.CompilerParams(use_tc_tiling_on_sc=False,
                                             needs_layout_passes=False),
        scratch_types=[
            pltpu.VMEM((C, H), jnp.float32),
            pltpu.VMEM((H,), jnp.float32),
            pltpu.VMEM((STEPS * TODC,), jnp.float32),
            pltpu.VMEM((4 * DHC,), jnp.float32),
            pltpu.VMEM((2, CH * C), jnp.float32),
            pltpu.VMEM((2, CH, H), jnp.float32),
            pltpu.VMEM((2, CH, TODC), jnp.float32),
            pltpu.VMEM((2, CH, DHC), jnp.float32),
            pltpu.VMEM((2, CH, ADPC), jnp.float32),
            pltpu.SemaphoreType.DMA((2,)),
            pltpu.SemaphoreType.DMA((2,)),
            pltpu.SemaphoreType.DMA((2,)),
            pltpu.SemaphoreType.DMA((2,)),
        ],
    )
    return run(x3, W_in, b_in, todf, dhf, adaptive_emb)


# speedup vs baseline: 3.9368x; 1.4449x over previous
import jax
import jax.numpy as jnp
from jax import lax
from jax.experimental import pallas as pl
from jax.experimental.pallas import tpu as pltpu
from jax.experimental.pallas import tpu_sc as plsc

B, T, N, C = 16, 12, 2048, 4
H = 80
TODC = 24
DHC = 8
ADPC = 40
OUTC = 152
STEPS = 288
CH = 128
NCH = N // CH
NW = 32
COMBOS = T * NCH
PER_W = COMBOS // NW
NK = PER_W * B
XSZ = CH * C


def _body(x3, w, b, todf, dhf, adp, out, w_v, b_v, tod_v, dh_v,
          x_v, hb, adp_v, sem_in, sem_out, sem_ain):
    cc = lax.axis_index("c")
    ss = lax.axis_index("s")
    wid = ss * 2 + cc

    pltpu.sync_copy(w, w_v)
    pltpu.sync_copy(b, b_v)
    pltpu.sync_copy(todf, tod_v)
    pltpu.sync_copy(dhf, dh_v)

    lane = lax.iota(jnp.int32, 16)
    cidx = [jnp.full((16,), i, jnp.int32) for i in range(8)]
    ws = [[w_v[d, pl.ds(j * 16, 16)] for j in range(5)] for d in range(4)]
    bs = [b_v[pl.ds(j * 16, 16)] for j in range(5)]

    def chunk_coords(k):
        ci = k // B
        bb = k % B
        cid = wid * PER_W + ci
        t = cid // NCH
        n0 = pl.multiple_of((cid % NCH) * CH, CH)
        return ci, bb, t, n0

    def in_descs(k):
        ci, bb, t, n0 = chunk_coords(k)
        slot = k % 2
        xs = pl.multiple_of(slot * XSZ, XSZ)
        rs = pl.multiple_of(slot * CH, CH)
        return [
            pltpu.make_async_copy(
                x3.at[bb, t, pl.ds(pl.multiple_of(n0 * C, XSZ), XSZ)],
                x_v.at[pl.ds(xs, XSZ)], sem_in.at[slot]),
            pltpu.make_async_copy(
                adp.at[t, pl.ds(n0, CH), :],
                adp_v.at[pl.ds(rs, CH), :], sem_ain.at[slot]),
        ]

    def issue_in(k):
        for d in in_descs(k):
            d.start()

    def out_desc(k):
        ci, bb, t, n0 = chunk_coords(k)
        slot = k % 2
        rs = pl.multiple_of(slot * CH, CH)
        return pltpu.make_async_copy(hb.at[pl.ds(rs, CH), :],
                                     out.at[bb, t, pl.ds(n0, CH), :],
                                     sem_out.at[slot])

    issue_in(0)

    def chunk_body(k, carry):
        ci, bb, t, n0 = chunk_coords(k)
        slot = k % 2

        xd, ad = in_descs(k)
        xd.wait()
        ad.wait()

        @pl.when(k + 1 < NK)
        def _():
            issue_in(k + 1)

        @pl.when(k >= 2)
        def _():
            out_desc(k - 2).wait()

        xbase = slot * XSZ
        rbase = slot * CH

        def h_body(p2):
            xx = x_v[pl.ds(pl.multiple_of(xbase + p2 * 8, 8), 16)]
            for pt in range(2):
                p = p2 * 2 + pt
                r = rbase + p
                bc = [jnp.take_along_axis(xx, cidx[pt * 4 + d], axis=0)
                      for d in range(4)]
                for j in range(5):
                    acc = bs[j] + bc[0] * ws[0][j]
                    acc = acc + bc[1] * ws[1][j]
                    acc = acc + bc[2] * ws[2][j]
                    acc = acc + bc[3] * ws[3][j]
                    hb[r, pl.ds(j * 16, 16)] = acc
                hb[r, pl.ds(112, 16)] = adp_v[r, pl.ds(0, 16)]
                hb[r, pl.ds(128, 16)] = adp_v[r, pl.ds(16, 16)]
                hb[r, pl.ds(136, 16)] = adp_v[r, pl.ds(24, 16)]
        plsc.parallel_loop(0, CH // 2, 1, unroll=2)(h_body)

        def g_body(g):
            rows = g * 16 + lane
            xoff = xbase + rows * C
            tod_f = plsc.load_gather(x_v, [xoff + 1])
            dow_f = plsc.load_gather(x_v, [xoff + 2])
            hol_f = plsc.load_gather(x_v, [xoff + 3])
            tod_i = jnp.clip((tod_f * float(STEPS)).astype(jnp.int32),
                             0, STEPS - 1)
            toff = tod_i * TODC
            dh_i = (jnp.clip(dow_f.astype(jnp.int32), 0, 1) * 2
                    + jnp.clip(hol_f.astype(jnp.int32), 0, 1))
            dhoff = dh_i * DHC
            hrows = rbase + rows
            for ch in range(TODC):
                v = plsc.load_gather(tod_v, [toff + ch])
                plsc.store_scatter(
                    hb, [hrows, jnp.full((16,), H + ch, jnp.int32)], v)
            for ch in range(DHC):
                v = plsc.load_gather(dh_v, [dhoff + ch])
                plsc.store_scatter(
                    hb, [hrows, jnp.full((16,), H + TODC + ch, jnp.int32)],
                    v)
        plsc.parallel_loop(0, CH // 16, 1, unroll=1)(g_body)

        out_desc(k).start()
        return carry

    lax.fori_loop(0, NK, chunk_body, 0)

    out_desc(NK - 2).wait()
    out_desc(NK - 1).wait()


def kernel(x, W_in, b_in, tod_table, dow_table, hol_table, adaptive_emb):
    x3 = x.reshape(B, T, N * C)
    todf = tod_table.reshape(-1)
    dhf = jnp.concatenate(
        [jnp.repeat(dow_table, 2, axis=0), jnp.tile(hol_table, (2, 1))],
        axis=1).reshape(-1)

    mesh = plsc.VectorSubcoreMesh(core_axis_name="c", subcore_axis_name="s")
    run = pl.kernel(
        _body,
        out_type=jax.ShapeDtypeStruct((B, T, N, OUTC), jnp.float32),
        mesh=mesh,
        compiler_params=pltpu.CompilerParams(use_tc_tiling_on_sc=True,
                                             needs_layout_passes=False),
        scratch_types=[
            pltpu.VMEM((C, H), jnp.float32),
            pltpu.VMEM((H,), jnp.float32),
            pltpu.VMEM((STEPS * TODC,), jnp.float32),
            pltpu.VMEM((4 * DHC,), jnp.float32),
            pltpu.VMEM((2 * XSZ,), jnp.float32),
            pltpu.VMEM((2 * CH, OUTC), jnp.float32),
            pltpu.VMEM((2 * CH, ADPC), jnp.float32),
            pltpu.SemaphoreType.DMA((2,)),
            pltpu.SemaphoreType.DMA((2,)),
            pltpu.SemaphoreType.DMA((2,)),
        ],
    )
    return run(x3, W_in, b_in, todf, dhf, adaptive_emb)
